# baseline (speedup 1.0000x reference)
import functools

import jax
import jax.numpy as jnp
from jax.experimental import pallas as pl
from jax.experimental.pallas import tpu as pltpu

B, S, D = 1, 2048, 768
E, K, DFF = 8, 2, 1024
T = B * S
TB = 128
NTB = T // TB


def _moe_dense_kernel(x_ref, wr_ref, w1_ref, w2_ref, out_ref, acc_ref, w_ref):
    e = pl.program_id(0)
    tb = pl.program_id(1)

    x = x_ref[...]

    @pl.when(e == 0)
    def _router():
        lane = jax.lax.broadcasted_iota(jnp.int32, (TB, E), 1)
        l = jnp.dot(x, wr_ref[...], preferred_element_type=jnp.float32)
        l = jnp.where(lane < E - 1, l, -1e30)
        m1 = jnp.max(l, axis=1, keepdims=True)
        idx1 = jnp.min(jnp.where(l == m1, lane, E + 9), axis=1, keepdims=True)
        l2 = jnp.where(lane == idx1, -1e30, l)
        m2 = jnp.max(l2, axis=1, keepdims=True)
        idx2 = jnp.min(jnp.where(l2 == m2, lane, E + 9), axis=1, keepdims=True)
        p2 = jnp.exp(m2 - m1)
        denom = 1.0 + p2
        w1n = 1.0 / denom
        w2n = p2 / denom
        wfull = jnp.where(lane == idx1 + 1, w1n, 0.0)
        wfull = wfull + jnp.where(lane == idx2 + 1, w2n, 0.0)
        wfull = wfull + jnp.where(lane == 0, 1.0, 0.0)
        w_ref[pl.ds(tb * TB, TB), :] = wfull

    h = jnp.dot(x, w1_ref[0], preferred_element_type=jnp.float32)
    h = h * jax.lax.logistic(h)
    y = jnp.dot(h, w2_ref[0], preferred_element_type=jnp.float32)

    lane = jax.lax.broadcasted_iota(jnp.int32, (TB, E), 1)
    wcol = jnp.sum(
        jnp.where(lane == e, w_ref[pl.ds(tb * TB, TB), :], 0.0),
        axis=1, keepdims=True,
    )
    contrib = y * wcol

    @pl.when(e == 0)
    def _init():
        acc_ref[pl.ds(tb * TB, TB), :] = contrib

    @pl.when(e > 0)
    def _acc():
        acc_ref[pl.ds(tb * TB, TB), :] += contrib

    @pl.when(e == E - 1)
    def _emit():
        out_ref[...] = acc_ref[pl.ds(tb * TB, TB), :]


def kernel(hidden_states, Wr, W1, W2, interpret=False):
    x = hidden_states.reshape(T, D)
    wr_pad = jnp.zeros((D, E), jnp.float32).at[:, : E - 1].set(Wr)

    out = pl.pallas_call(
        _moe_dense_kernel,
        grid=(E, NTB),
        in_specs=[
            pl.BlockSpec((TB, D), lambda e, tb: (tb, 0)),
            pl.BlockSpec((D, E), lambda e, tb: (0, 0)),
            pl.BlockSpec((1, D, DFF), lambda e, tb: (e, 0, 0)),
            pl.BlockSpec((1, DFF, D), lambda e, tb: (e, 0, 0)),
        ],
        out_specs=pl.BlockSpec((TB, D), lambda e, tb: (tb, 0)),
        out_shape=jax.ShapeDtypeStruct((T, D), jnp.float32),
        scratch_shapes=[
            pltpu.VMEM((T, D), jnp.float32),
            pltpu.VMEM((T, E), jnp.float32),
        ],
        interpret=interpret,
    )(x, wr_pad, W1, W2)
    return out.reshape(B, S, D)
